# baseline (speedup 1.0000x reference)
import functools

import jax
import jax.numpy as jnp
from jax import lax
from jax.experimental import pallas as pl
from jax.experimental.pallas import tpu as pltpu
from jax.experimental.pallas import tpu_sc as plsc

B = 1024
EMB = 64
VOCAB = 100000

VT = 2048
NT = (VOCAB + VT - 1) // VT
NEG = -1e30


def _sc_gather(idx2, table2):
    info = plsc.get_sparse_core_info()
    nw = info.num_cores * info.num_subcores
    bpw = B // nw
    mesh = plsc.VectorSubcoreMesh(core_axis_name="c", subcore_axis_name="s")

    @functools.partial(
        pl.kernel,
        mesh=mesh,
        out_type=jax.ShapeDtypeStruct((B, 2 * EMB), jnp.float32),
        scratch_types=[
            pltpu.VMEM((bpw,), jnp.int32),
            pltpu.VMEM((bpw, 2 * EMB), jnp.float32),
            pltpu.SemaphoreType.DMA,
        ],
    )
    def gather_kernel(idx_hbm, table_hbm, out_hbm, idx_v, rows_v, sem):
        wid = lax.axis_index("s") * info.num_cores + lax.axis_index("c")
        base = wid * bpw
        pltpu.sync_copy(idx_hbm.at[pl.ds(base, bpw)], idx_v)
        pltpu.async_copy(table_hbm.at[idx_v], rows_v, sem).wait()
        pltpu.sync_copy(rows_v, out_hbm.at[pl.ds(base, bpw)])

    return gather_kernel(idx2, table2)


def _tc_body(h2_ref, par_ref, w_ref, b_ref, out_ref, m_ref, s_ref, lse_ref):
    phase = pl.program_id(0)
    j = pl.program_id(1)

    hsel = jnp.where(par_ref[...] == 0,
                     h2_ref[:, :EMB], h2_ref[:, EMB:])
    h = jnp.maximum(hsel, 0.0)
    logits = lax.dot_general(
        h, w_ref[...], (((1,), (1,)), ((), ())),
        preferred_element_type=jnp.float32,
    ) + b_ref[...]

    @pl.when(phase == 0)
    def _stats():
        col = j * VT + lax.broadcasted_iota(jnp.int32, (1, VT), 1)
        masked = jnp.where(col < VOCAB, logits, NEG)

        @pl.when(j == 0)
        def _init():
            m_ref[...] = jnp.full_like(m_ref, NEG)
            s_ref[...] = jnp.zeros_like(s_ref)

        m_old = m_ref[...]
        m_new = jnp.maximum(m_old, jnp.max(masked, axis=1, keepdims=True))
        s_ref[...] = (s_ref[...] * jnp.exp(m_old - m_new)
                      + jnp.sum(jnp.exp(masked - m_new), axis=1, keepdims=True))
        m_ref[...] = m_new

        @pl.when(j == pl.num_programs(1) - 1)
        def _finalize():
            lse_ref[...] = m_ref[...] + jnp.log(s_ref[...])

    @pl.when(phase == 1)
    def _write():
        out_ref[...] = logits - lse_ref[...]


def _tc_logsoftmax(h2, par, W, b2d):
    return pl.pallas_call(
        _tc_body,
        grid=(2, NT),
        in_specs=[
            pl.BlockSpec((B, 2 * EMB), lambda p, j: (0, 0)),
            pl.BlockSpec((B, 1), lambda p, j: (0, 0)),
            pl.BlockSpec((VT, EMB), lambda p, j: (j, 0)),
            pl.BlockSpec((1, VT), lambda p, j: (0, j)),
        ],
        out_specs=pl.BlockSpec((B, VT), lambda p, j: (0, j * p)),
        out_shape=jax.ShapeDtypeStruct((B, VOCAB), jnp.float32),
        scratch_shapes=[
            pltpu.VMEM((B, 1), jnp.float32),
            pltpu.VMEM((B, 1), jnp.float32),
            pltpu.VMEM((B, 1), jnp.float32),
        ],
    )(h2, par, W, b2d)


def kernel(input, table, W, b):
    idx = input.astype(jnp.int32)
    table2 = table.reshape(VOCAB // 2, 2 * EMB)
    h2 = _sc_gather(idx >> 1, table2)
    par = (idx & 1).astype(jnp.float32).reshape(B, 1)
    return _tc_logsoftmax(h2, par, W, b.reshape(1, VOCAB))


# speedup vs baseline: 1.0521x; 1.0521x over previous
import functools

import jax
import jax.numpy as jnp
from jax import lax
from jax.experimental import pallas as pl
from jax.experimental.pallas import tpu as pltpu
from jax.experimental.pallas import tpu_sc as plsc

B = 1024
EMB = 64
VOCAB = 100000

VT = 2048
NT = (VOCAB + VT - 1) // VT
NEG = -1e30


def _sc_gather(idx2, table2):
    info = plsc.get_sparse_core_info()
    nw = info.num_cores * info.num_subcores
    bpw = B // nw
    mesh = plsc.VectorSubcoreMesh(core_axis_name="c", subcore_axis_name="s")

    @functools.partial(
        pl.kernel,
        mesh=mesh,
        out_type=jax.ShapeDtypeStruct((B, 2 * EMB), jnp.float32),
        scratch_types=[
            pltpu.VMEM((bpw,), jnp.int32),
            pltpu.VMEM((bpw, 2 * EMB), jnp.float32),
            pltpu.SemaphoreType.DMA,
        ],
    )
    def gather_kernel(idx_hbm, table_hbm, out_hbm, idx_v, rows_v, sem):
        wid = lax.axis_index("s") * info.num_cores + lax.axis_index("c")
        base = wid * bpw
        pltpu.sync_copy(idx_hbm.at[pl.ds(base, bpw)], idx_v)
        pltpu.async_copy(table_hbm.at[idx_v], rows_v, sem).wait()
        pltpu.sync_copy(rows_v, out_hbm.at[pl.ds(base, bpw)])

    return gather_kernel(idx2, table2)


def _tc_body(h2_ref, par_ref, w_ref, b_ref, out_ref,
             hs_ref, m_ref, s_ref, lse_ref):
    phase = pl.program_id(0)
    j = pl.program_id(1)

    @pl.when((phase == 0) & (j == 0))
    def _prep():
        hsel = jnp.where(par_ref[...] == 0,
                         h2_ref[:, :EMB], h2_ref[:, EMB:])
        hs_ref[...] = jnp.maximum(hsel, 0.0)
        m_ref[...] = jnp.full_like(m_ref, NEG)
        s_ref[...] = jnp.zeros_like(s_ref)

    logits = lax.dot_general(
        hs_ref[...], w_ref[...], (((1,), (1,)), ((), ())),
        preferred_element_type=jnp.float32,
    ) + b_ref[...]

    @pl.when(phase == 0)
    def _stats():
        last = pl.num_programs(1) - 1

        def acc(vals):
            m_old = m_ref[...]
            m_new = jnp.maximum(m_old, jnp.max(vals, axis=1, keepdims=True))
            s_ref[...] = (s_ref[...] * jnp.exp(m_old - m_new)
                          + jnp.sum(jnp.exp(vals - m_new), axis=1,
                                    keepdims=True))
            m_ref[...] = m_new

        @pl.when(j < last)
        def _interior():
            acc(logits)

        @pl.when(j == last)
        def _tail():
            col = j * VT + lax.broadcasted_iota(jnp.int32, (1, VT), 1)
            acc(jnp.where(col < VOCAB, logits, NEG))
            lse_ref[...] = m_ref[...] + jnp.log(s_ref[...])

    @pl.when(phase == 1)
    def _write():
        out_ref[...] = logits - lse_ref[...]


def _tc_logsoftmax(h2, par, W, b2d):
    return pl.pallas_call(
        _tc_body,
        grid=(2, NT),
        in_specs=[
            pl.BlockSpec((B, 2 * EMB), lambda p, j: (0, 0)),
            pl.BlockSpec((B, 1), lambda p, j: (0, 0)),
            pl.BlockSpec((VT, EMB), lambda p, j: (j, 0)),
            pl.BlockSpec((1, VT), lambda p, j: (0, j)),
        ],
        out_specs=pl.BlockSpec((B, VT), lambda p, j: (0, j * p)),
        out_shape=jax.ShapeDtypeStruct((B, VOCAB), jnp.float32),
        scratch_shapes=[
            pltpu.VMEM((B, EMB), jnp.float32),
            pltpu.VMEM((B, 1), jnp.float32),
            pltpu.VMEM((B, 1), jnp.float32),
            pltpu.VMEM((B, 1), jnp.float32),
        ],
    )(h2, par, W, b2d)


def kernel(input, table, W, b):
    idx = input.astype(jnp.int32)
    table2 = table.reshape(VOCAB // 2, 2 * EMB)
    h2 = _sc_gather(idx >> 1, table2)
    par = (idx & 1).astype(jnp.float32).reshape(B, 1)
    return _tc_logsoftmax(h2, par, W, b.reshape(1, VOCAB))


# speedup vs baseline: 1.0629x; 1.0102x over previous
import functools

import jax
import jax.numpy as jnp
from jax import lax
from jax.experimental import pallas as pl
from jax.experimental.pallas import tpu as pltpu
from jax.experimental.pallas import tpu_sc as plsc

B = 1024
EMB = 64
VOCAB = 100000

VT = 2048
NT = (VOCAB + VT - 1) // VT
NEG = -1e30


def _sc_gather(idx2, table2):
    info = plsc.get_sparse_core_info()
    nw = info.num_cores * info.num_subcores
    bpw = B // nw
    mesh = plsc.VectorSubcoreMesh(core_axis_name="c", subcore_axis_name="s")

    @functools.partial(
        pl.kernel,
        mesh=mesh,
        out_type=jax.ShapeDtypeStruct((B, 2 * EMB), jnp.float32),
        scratch_types=[
            pltpu.VMEM((bpw,), jnp.int32),
            pltpu.VMEM((bpw, 2 * EMB), jnp.float32),
            pltpu.SemaphoreType.DMA,
        ],
    )
    def gather_kernel(idx_hbm, table_hbm, out_hbm, idx_v, rows_v, sem):
        wid = lax.axis_index("s") * info.num_cores + lax.axis_index("c")
        base = wid * bpw
        pltpu.sync_copy(idx_hbm.at[pl.ds(base, bpw)], idx_v)
        pltpu.async_copy(table_hbm.at[idx_v], rows_v, sem).wait()
        pltpu.sync_copy(rows_v, out_hbm.at[pl.ds(base, bpw)])

    return gather_kernel(idx2, table2)


def _tc_body(h2_ref, par_ref, w_ref, b_ref, out_ref,
             hs_ref, m_ref, s_ref, lse_ref):
    phase = pl.program_id(0)
    j = pl.program_id(1)

    @pl.when((phase == 0) & (j == 0))
    def _prep():
        hsel = jnp.where(par_ref[...] == 0,
                         h2_ref[:, :EMB], h2_ref[:, EMB:])
        hs_ref[...] = jnp.maximum(hsel, 0.0).astype(jnp.bfloat16)
        m_ref[...] = jnp.full_like(m_ref, NEG)
        s_ref[...] = jnp.zeros_like(s_ref)

    logits = lax.dot_general(
        hs_ref[...], w_ref[...], (((1,), (1,)), ((), ())),
        preferred_element_type=jnp.float32,
    ) + b_ref[...]

    @pl.when(phase == 0)
    def _stats():
        last = pl.num_programs(1) - 1

        def acc(vals):
            m_old = m_ref[...]
            m_new = jnp.maximum(m_old, jnp.max(vals, axis=1, keepdims=True))
            s_ref[...] = (s_ref[...] * jnp.exp(m_old - m_new)
                          + jnp.sum(jnp.exp(vals - m_new), axis=1,
                                    keepdims=True))
            m_ref[...] = m_new

        @pl.when(j < last)
        def _interior():
            acc(logits)

        @pl.when(j == last)
        def _tail():
            col = j * VT + lax.broadcasted_iota(jnp.int32, (1, VT), 1)
            acc(jnp.where(col < VOCAB, logits, NEG))
            lse_ref[...] = m_ref[...] + jnp.log(s_ref[...])

    @pl.when(phase == 1)
    def _write():
        out_ref[...] = logits - lse_ref[...]


def _tc_logsoftmax(h2, par, W, b2d):
    return pl.pallas_call(
        _tc_body,
        grid=(2, NT),
        in_specs=[
            pl.BlockSpec((B, 2 * EMB), lambda p, j: (0, 0)),
            pl.BlockSpec((B, 1), lambda p, j: (0, 0)),
            pl.BlockSpec((VT, EMB), lambda p, j: (j, 0)),
            pl.BlockSpec((1, VT), lambda p, j: (0, j)),
        ],
        out_specs=pl.BlockSpec((B, VT), lambda p, j: (0, j * p)),
        out_shape=jax.ShapeDtypeStruct((B, VOCAB), jnp.float32),
        scratch_shapes=[
            pltpu.VMEM((B, EMB), jnp.bfloat16),
            pltpu.VMEM((B, 1), jnp.float32),
            pltpu.VMEM((B, 1), jnp.float32),
            pltpu.VMEM((B, 1), jnp.float32),
        ],
    )(h2, par, W, b2d)


def kernel(input, table, W, b):
    idx = input.astype(jnp.int32)
    table2 = table.reshape(VOCAB // 2, 2 * EMB)
    h2 = _sc_gather(idx >> 1, table2)
    par = (idx & 1).astype(jnp.float32).reshape(B, 1)
    return _tc_logsoftmax(h2, par, W.astype(jnp.bfloat16),
                          b.reshape(1, VOCAB))
